# baseline (speedup 1.0000x reference)
import jax
import jax.numpy as jnp
from jax import lax
from jax.experimental import pallas as pl
from jax.experimental.pallas import tpu as pltpu

EPS = 1e-11
LOG2E = 1.4426950408889634
LN2 = 0.6931471805599453

ROWS = 128
COLS = 100000
BSA = 1000
NBA = COLS // BSA
XLOOK = 12
UK = 10
BSB = 1000
NBB = COLS // BSB
OK_ = 8
NEG_INF = -3.0e38


def _gs_kernel(x_hbm, u_hbm, o_hbm, y_scr, ubuf, obuf, x_sem, u_sem, o_sem):
    def xcopy(c):
        return pltpu.make_async_copy(
            x_hbm.at[pl.ds(c * BSA, BSA), :],
            y_scr.at[pl.ds(c * BSA, BSA), :],
            x_sem.at[c],
        )

    def ucopy(c, slot):
        return pltpu.make_async_copy(
            u_hbm.at[pl.ds(c * BSA, BSA), :], ubuf.at[slot], u_sem.at[slot]
        )

    def ocopy(c, slot):
        return pltpu.make_async_copy(
            obuf.at[slot], o_hbm.at[pl.ds(c * BSB, BSB), :], o_sem.at[slot]
        )

    for c in range(XLOOK):
        xcopy(c).start()
    for c in range(UK):
        ucopy(c, c).start()

    def body_a(c, carry):
        m, s = carry
        slot = lax.rem(c, UK)
        xcopy(c).wait()
        ucopy(c, slot).wait()
        xb = y_scr[pl.ds(c * BSA, BSA), :]
        ub = ubuf[slot]
        y2 = xb * LOG2E - jnp.log2(-LN2 * jnp.log2(ub + EPS))
        y_scr[pl.ds(c * BSA, BSA), :] = y2
        bm = jnp.max(y2, axis=0, keepdims=True)
        m_new = jnp.maximum(m, bm)
        bs = jnp.sum(jnp.exp2(y2 - m_new), axis=0, keepdims=True)
        s = s * jnp.exp2(m - m_new) + bs

        @pl.when(c + XLOOK < NBA)
        def _():
            xcopy(c + XLOOK).start()

        @pl.when(c + UK < NBA)
        def _():
            ucopy(c + UK, slot).start()

        return m_new, s

    m0 = jnp.full((1, ROWS), NEG_INF, jnp.float32)
    s0 = jnp.zeros((1, ROWS), jnp.float32)
    m, s = lax.fori_loop(0, NBA, body_a, (m0, s0))
    inv = 1.0 / s

    def body_b(c, _):
        slot = lax.rem(c, OK_)

        @pl.when(c >= OK_)
        def _():
            ocopy(c - OK_, slot).wait()

        e = jnp.exp2(y_scr[pl.ds(c * BSB, BSB), :] - m) * inv
        obuf[slot] = e
        ocopy(c, slot).start()
        return 0

    lax.fori_loop(0, NBB, body_b, 0)
    for c in range(NBB - OK_, NBB):
        ocopy(c, c % OK_).wait()


def kernel(logits, u):
    xt = logits.T
    ut = u.T
    hbm = pl.BlockSpec(memory_space=pltpu.MemorySpace.HBM)
    out = pl.pallas_call(
        _gs_kernel,
        in_specs=[hbm, hbm],
        out_specs=hbm,
        out_shape=jax.ShapeDtypeStruct((COLS, ROWS), jnp.float32),
        scratch_shapes=[
            pltpu.VMEM((COLS, ROWS), jnp.float32),
            pltpu.VMEM((UK, BSA, ROWS), jnp.float32),
            pltpu.VMEM((OK_, BSB, ROWS), jnp.float32),
            pltpu.SemaphoreType.DMA((NBA,)),
            pltpu.SemaphoreType.DMA((UK,)),
            pltpu.SemaphoreType.DMA((OK_,)),
        ],
    )(xt, ut)
    return out.T


# speedup vs baseline: 1.1214x; 1.1214x over previous
import jax
import jax.numpy as jnp
from jax import lax
from jax.experimental import pallas as pl
from jax.experimental.pallas import tpu as pltpu

EPS = 1e-11

ROWS = 128
COLS = 100000
BSA = 1000
NBA = COLS // BSA
XLOOK = 12
UK = 10
BSB = 1000
NBB = COLS // BSB
OK_ = 8
NEG_INF = -3.0e38


def _gs_kernel(x_hbm, u_hbm, o_hbm, y_scr, ubuf, obuf, mblk, x_sem, u_sem,
               o_sem):
    def xcopy(c):
        return pltpu.make_async_copy(
            x_hbm.at[pl.ds(c * BSA, BSA), :],
            y_scr.at[pl.ds(c * BSA, BSA), :],
            x_sem.at[c],
        )

    def ucopy(c, slot):
        return pltpu.make_async_copy(
            u_hbm.at[pl.ds(c * BSA, BSA), :], ubuf.at[slot], u_sem.at[slot]
        )

    def ocopy(c, slot):
        return pltpu.make_async_copy(
            obuf.at[slot], o_hbm.at[pl.ds(c * BSB, BSB), :], o_sem.at[slot]
        )

    for c in range(XLOOK):
        xcopy(c).start()
    for c in range(UK):
        ucopy(c, c).start()

    def body_a(c, carry):
        m, s = carry
        slot = lax.rem(c, UK)
        xcopy(c).wait()
        ucopy(c, slot).wait()
        xb = y_scr[pl.ds(c * BSA, BSA), :]
        ub = ubuf[slot]
        y = xb - jnp.log(-jnp.log(ub + EPS))
        bm = jnp.max(y, axis=0, keepdims=True)
        m_new = jnp.maximum(m, bm)
        ee = jnp.exp(y - m_new)
        y_scr[pl.ds(c * BSA, BSA), :] = ee
        mblk[pl.ds(c * 8, 8), :] = jnp.broadcast_to(m_new, (8, ROWS))
        bs = jnp.sum(ee, axis=0, keepdims=True)
        s = s * jnp.exp(m - m_new) + bs

        @pl.when(c + XLOOK < NBA)
        def _():
            xcopy(c + XLOOK).start()

        @pl.when(c + UK < NBA)
        def _():
            ucopy(c + UK, slot).start()

        return m_new, s

    m0 = jnp.full((1, ROWS), NEG_INF, jnp.float32)
    s0 = jnp.zeros((1, ROWS), jnp.float32)
    m, s = lax.fori_loop(0, NBA, body_a, (m0, s0))
    inv = 1.0 / s

    def body_b(c, _):
        slot = lax.rem(c, OK_)

        @pl.when(c >= OK_)
        def _():
            ocopy(c - OK_, slot).wait()

        mb = mblk[pl.ds(c * 8, 8), :][0:1]
        f = jnp.exp(mb - m) * inv
        obuf[slot] = y_scr[pl.ds(c * BSB, BSB), :] * f
        ocopy(c, slot).start()
        return 0

    lax.fori_loop(0, NBB, body_b, 0)
    for c in range(NBB - OK_, NBB):
        ocopy(c, c % OK_).wait()


def kernel(logits, u):
    xt = logits.T
    ut = u.T
    hbm = pl.BlockSpec(memory_space=pltpu.MemorySpace.HBM)
    out = pl.pallas_call(
        _gs_kernel,
        in_specs=[hbm, hbm],
        out_specs=hbm,
        out_shape=jax.ShapeDtypeStruct((COLS, ROWS), jnp.float32),
        scratch_shapes=[
            pltpu.VMEM((COLS, ROWS), jnp.float32),
            pltpu.VMEM((UK, BSA, ROWS), jnp.float32),
            pltpu.VMEM((OK_, BSB, ROWS), jnp.float32),
            pltpu.VMEM((NBA * 8, ROWS), jnp.float32),
            pltpu.SemaphoreType.DMA((NBA,)),
            pltpu.SemaphoreType.DMA((UK,)),
            pltpu.SemaphoreType.DMA((OK_,)),
        ],
    )(xt, ut)
    return out.T
